# baseline (speedup 1.0000x reference)
import jax
import jax.numpy as jnp
from jax import lax
from jax.experimental import pallas as pl
from jax.experimental.pallas import tpu as pltpu
from jax.experimental.pallas import tpu_sc as plsc

IGNORE = 1
L = 16
NC, NS = 2, 16
NW = NC * NS


def _make_sc_kernel(N, C):
    RPW = N // NW
    ZROWS = 8
    NCHUNK = RPW // ZROWS

    def body(g_hbm, t_hbm, out_hbm, zbuf, tloc, gloc, vals, fsems, vsem):
        wid = lax.axis_index("s") * NC + lax.axis_index("c")
        base = wid * RPW

        pltpu.sync_copy(t_hbm.at[pl.ds(base, RPW)], tloc)
        pltpu.sync_copy(g_hbm.at[pl.ds(base, RPW)], gloc)

        zeros16 = jnp.zeros((L,), jnp.float32)
        UNROLL = 8

        def zinit(i, carry):
            for r in range(ZROWS):
                for u in range(UNROLL):
                    off = pl.multiple_of(i * (L * UNROLL), L * UNROLL) + u * L
                    zbuf[r, pl.ds(off, L)] = zeros16
            return carry

        lax.fori_loop(0, C // (L * UNROLL), zinit, 0)

        W = 128

        def vzinit(i, carry):
            for u in range(W // L):
                vals[i, pl.ds(u * L, L)] = zeros16
            return carry

        lax.fori_loop(0, RPW, vzinit, 0)

        iota16 = lax.iota(jnp.int32, L)

        NSEM = 8
        DEPTH = 4

        def fill(c):
            row = pl.multiple_of(base + c * ZROWS, ZROWS)
            return pltpu.make_async_copy(
                zbuf, out_hbm.at[pl.ds(row, ZROWS)], fsems[c % NSEM])

        for c in range(2 * DEPTH):
            fill(c).start()

        GR = L // ZROWS
        NG = RPW // L

        def vwait():
            pltpu.make_async_copy(
                vals.at[pl.ds(0, 1), pl.ds(0, W)],
                out_hbm.at[pl.ds(base, 1), pl.ds(0, W)],
                vsem,
            ).wait()

        for g in range(NG):
            for k in range(GR):
                fill(g * GR + k).wait()
            if g + DEPTH < NG:
                for k in range(GR):
                    fill((g + DEPTH) * GR + k).start()
            off = g * L
            t16 = tloc[pl.ds(off, L)]
            g16 = gloc[pl.ds(off, L)]
            for r in range(L):
                i = off + r
                t_r = t16[r]
                val = jnp.where(t_r == IGNORE, 0.0, -g16[r])
                win = (t_r // W) * W
                seg = ((t_r - win) // L) * L
                vals[i, pl.ds(seg, L)] = jnp.where(
                    iota16 == t_r - win - seg, val, 0.0)
                pltpu.make_async_copy(
                    vals.at[pl.ds(i, 1), pl.ds(0, W)],
                    out_hbm.at[pl.ds(base + i, 1), pl.ds(win, W)],
                    vsem,
                ).start()

            if g >= 2:
                for _ in range(L):
                    vwait()

        for _ in range(2 * L):
            vwait()

    mesh = plsc.VectorSubcoreMesh(core_axis_name="c", subcore_axis_name="s")
    return pl.kernel(
        body,
        out_type=jax.ShapeDtypeStruct((N, C), jnp.float32),
        mesh=mesh,
        compiler_params=pltpu.CompilerParams(needs_layout_passes=False),
        scratch_types=[
            pltpu.VMEM((ZROWS, C), jnp.float32),
            pltpu.VMEM((RPW,), jnp.int32),
            pltpu.VMEM((RPW,), jnp.float32),
            pltpu.VMEM((RPW, 128), jnp.float32),
            [pltpu.SemaphoreType.DMA] * 8,
            pltpu.SemaphoreType.DMA,
        ],
    )


def kernel(grad_output, input, target, total_weight):
    N, C = input.shape
    tgt = target.astype(jnp.int32)
    return _make_sc_kernel(N, C)(grad_output, tgt)


# speedup vs baseline: 1.1313x; 1.1313x over previous
import jax
import jax.numpy as jnp
from jax import lax
from jax.experimental import pallas as pl
from jax.experimental.pallas import tpu as pltpu
from jax.experimental.pallas import tpu_sc as plsc

IGNORE = 1
L = 16
NC, NS = 2, 16
NW = NC * NS


def _make_sc_kernel(N, C):
    RPW = N // NW
    ZROWS = 4
    NCHUNK = RPW // ZROWS
    GRP = L // ZROWS

    def body(g_hbm, t_hbm, out_hbm, buf0, buf1, buf2, tloc, gloc,
             sem0, sem1, sem2):
        wid = lax.axis_index("s") * NC + lax.axis_index("c")
        base = wid * RPW

        pltpu.sync_copy(t_hbm.at[pl.ds(base, RPW)], tloc)
        pltpu.sync_copy(g_hbm.at[pl.ds(base, RPW)], gloc)

        zeros16 = jnp.zeros((L,), jnp.float32)
        iota16 = lax.iota(jnp.int32, L)
        UNROLL = 8

        def zinit(i, carry):
            for r in range(ZROWS):
                for u in range(UNROLL):
                    off = pl.multiple_of(i * (L * UNROLL), L * UNROLL) + u * L
                    buf0[r, pl.ds(off, L)] = zeros16
                    buf1[r, pl.ds(off, L)] = zeros16
                    buf2[r, pl.ds(off, L)] = zeros16
            return carry

        lax.fori_loop(0, C // (L * UNROLL), zinit, 0)

        def chunk_vectors(c):
            grp = c // GRP
            sub = c % GRP
            t16 = tloc[pl.ds(grp * L, L)]
            g16 = gloc[pl.ds(grp * L, L)]
            val16 = jnp.where(t16 == IGNORE, zeros16, -g16)
            ridx16 = iota16 - ZROWS * sub
            mask16 = (iota16 >= ZROWS * sub) & (iota16 < ZROWS * (sub + 1))
            return t16, val16, ridx16, mask16

        def dma(b, sem, c):
            row = pl.multiple_of(base + c * ZROWS, ZROWS)
            return pltpu.make_async_copy(b, out_hbm.at[pl.ds(row, ZROWS)], sem)

        NBUF = 3

        def do_chunk(c, b, sem):
            @pl.when(c >= NBUF)
            def _():
                dma(b, sem, c - NBUF).wait()
                pt16, _, pr16, pm16 = chunk_vectors(c - NBUF)
                plsc.store_scatter(b, [pr16, pt16], zeros16, mask=pm16)

            t16, val16, ridx16, mask16 = chunk_vectors(c)
            plsc.store_scatter(b, [ridx16, t16], val16, mask=mask16)
            dma(b, sem, c).start()

        def triple(p, carry):
            do_chunk(3 * p, buf0, sem0)
            do_chunk(3 * p + 1, buf1, sem1)
            do_chunk(3 * p + 2, buf2, sem2)
            return carry

        lax.fori_loop(0, NCHUNK // NBUF, triple, 0)
        do_chunk(NCHUNK - 1, buf0, sem0)

        dma(buf1, sem1, NCHUNK - 3).wait()
        dma(buf2, sem2, NCHUNK - 2).wait()
        dma(buf0, sem0, NCHUNK - 1).wait()

    mesh = plsc.VectorSubcoreMesh(core_axis_name="c", subcore_axis_name="s")
    return pl.kernel(
        body,
        out_type=jax.ShapeDtypeStruct((N, C), jnp.float32),
        mesh=mesh,
        compiler_params=pltpu.CompilerParams(needs_layout_passes=False),
        scratch_types=[
            pltpu.VMEM((ZROWS, C), jnp.float32),
            pltpu.VMEM((ZROWS, C), jnp.float32),
            pltpu.VMEM((ZROWS, C), jnp.float32),
            pltpu.VMEM((RPW,), jnp.int32),
            pltpu.VMEM((RPW,), jnp.float32),
            pltpu.SemaphoreType.DMA,
            pltpu.SemaphoreType.DMA,
            pltpu.SemaphoreType.DMA,
        ],
    )


def kernel(grad_output, input, target, total_weight):
    N, C = input.shape
    tgt = target.astype(jnp.int32)
    return _make_sc_kernel(N, C)(grad_output, tgt)
